# baseline (speedup 1.0000x reference)
import functools

import jax
import jax.numpy as jnp
from jax import lax
from jax.experimental import pallas as pl
from jax.experimental.pallas import tpu as pltpu
from jax.experimental.pallas import tpu_sc as plsc

_T, _D = 2048, 1024
_E, _TOPK = 8, 2
_I = 512
_SI = 512
_P = _T * _TOPK
_BLK = 128
_NB = _P // _BLK + _E
_L = _NB * _BLK
_NW = 32
_TB = 256


def _dispatch_body(x_ref, wr_ref, probs_ref, slot_ref, st_ref, sw_ref,
                   bexp_ref, rank_ref, ep_ref, wp_ref):
    x = x_ref[...]
    logits = lax.dot_general(x, wr_ref[...], (((1,), (1,)), ((), ())),
                             preferred_element_type=jnp.float32)
    m = jnp.max(logits, axis=-1, keepdims=True)
    ex = jnp.exp(logits - m)
    probs = ex / jnp.sum(ex, axis=-1, keepdims=True)
    probs_ref[...] = probs

    iota_e = lax.broadcasted_iota(jnp.int32, (_T, _E), 1)
    i1 = jnp.argmax(probs, axis=-1)[:, None]
    oh1 = (iota_e == i1)
    m1 = jnp.max(probs, axis=-1, keepdims=True)
    masked = jnp.where(oh1, -jnp.inf, probs)
    i2 = jnp.argmax(masked, axis=-1)[:, None]
    m2 = jnp.max(masked, axis=-1, keepdims=True)
    denom = m1 + m2 + 1e-9

    ep_ref[0:_T, :] = i1
    ep_ref[_T:_P, :] = i2
    wp_ref[0:_T, :] = m1 / denom
    wp_ref[_T:_P, :] = m2 / denom
    ep = ep_ref[...]

    iota_e8 = lax.broadcasted_iota(jnp.int32, (1, _E), 1)
    tri = (lax.broadcasted_iota(jnp.int32, (_BLK, _BLK), 0)
           >= lax.broadcasted_iota(jnp.int32, (_BLK, _BLK), 1)
           ).astype(jnp.float32)

    def cum_body(c, carry):
        ep_c = ep_ref[pl.ds(c * _BLK, _BLK), :]
        a_c = (ep_c == iota_e8).astype(jnp.float32)
        c_c = lax.dot_general(tri, a_c, (((1,), (0,)), ((), ())),
                              preferred_element_type=jnp.float32) + carry
        rank_c = jnp.sum(c_c * a_c, axis=1, keepdims=True) - 1.0
        rank_ref[pl.ds(c * _BLK, _BLK), :] = rank_c
        return lax.slice(c_c, (_BLK - 1, 0), (_BLK, _E))

    counts = lax.fori_loop(0, _P // _BLK, cum_body,
                           jnp.zeros((1, _E), jnp.float32))

    pcnt = jnp.floor((counts + (_BLK - 1)) / _BLK) * _BLK
    below = (lax.broadcasted_iota(jnp.int32, (_E, _E), 0)
             < lax.broadcasted_iota(jnp.int32, (_E, _E), 1)).astype(jnp.float32)
    off = lax.dot_general(pcnt, below, (((1,), (0,)), ((), ())),
                          preferred_element_type=jnp.float32)
    bend = (off + pcnt) / _BLK
    iota_b = lax.broadcasted_iota(jnp.int32, (64, 1), 0).astype(jnp.float32)
    bexp = jnp.sum((bend <= iota_b).astype(jnp.int32), axis=1, keepdims=True)
    bexp_ref[...] = jnp.minimum(bexp, _E - 1)

    off_g = jnp.sum(jnp.where(ep == iota_e8, off, 0.0), axis=1, keepdims=True)
    slot = off_g + rank_ref[...]
    slot_ref[...] = slot.astype(jnp.int32)

    iota_j = lax.broadcasted_iota(jnp.int32, (_BLK, 1), 0)
    iota_l = lax.broadcasted_iota(jnp.int32, (1, _L), 1)

    def sc_body(c, carry):
        st_acc, sw_acc = carry
        slot_c = slot_ref[pl.ds(c * _BLK, _BLK), :]
        p_c = iota_j + c * _BLK
        tok_c = jnp.where(p_c >= _T, p_c - _T, p_c).astype(jnp.float32)
        wp_c = wp_ref[pl.ds(c * _BLK, _BLK), :]
        hit = (slot_c == iota_l)
        st_acc = st_acc + jnp.sum(jnp.where(hit, tok_c, 0.0), axis=0,
                                  keepdims=True)
        sw_acc = sw_acc + jnp.sum(jnp.where(hit, wp_c, 0.0), axis=0,
                                  keepdims=True)
        return st_acc, sw_acc

    st_acc, sw_acc = lax.fori_loop(
        0, _P // _BLK, sc_body,
        (jnp.zeros((1, _L), jnp.float32), jnp.zeros((1, _L), jnp.float32)))
    st_ref[...] = st_acc.astype(jnp.int32)
    sw_ref[...] = sw_acc


def _dispatch(x, router_weight):
    return pl.pallas_call(
        _dispatch_body,
        grid=(1,),
        in_specs=[
            pl.BlockSpec((_T, _D), lambda i: (0, 0)),
            pl.BlockSpec((_E, _D), lambda i: (0, 0)),
        ],
        out_specs=[
            pl.BlockSpec((_T, _E), lambda i: (0, 0)),
            pl.BlockSpec((_P, 1), lambda i: (0, 0)),
            pl.BlockSpec((1, _L), lambda i: (0, 0)),
            pl.BlockSpec((1, _L), lambda i: (0, 0)),
            pl.BlockSpec((64, 1), lambda i: (0, 0)),
        ],
        out_shape=[
            jax.ShapeDtypeStruct((_T, _E), jnp.float32),
            jax.ShapeDtypeStruct((_P, 1), jnp.int32),
            jax.ShapeDtypeStruct((1, _L), jnp.int32),
            jax.ShapeDtypeStruct((1, _L), jnp.float32),
            jax.ShapeDtypeStruct((64, 1), jnp.int32),
        ],
        scratch_shapes=[pltpu.VMEM((_P, 1), jnp.float32),
                        pltpu.VMEM((_P, 1), jnp.int32),
                        pltpu.VMEM((_P, 1), jnp.float32)],
    )(x, router_weight)


_ROWS_W = _L // _NW
_GCH = 80
_DP = _D // 2


def _sc_gather(xp, st_flat):
    mesh = plsc.VectorSubcoreMesh(core_axis_name="c", subcore_axis_name="s")

    @functools.partial(
        pl.kernel, mesh=mesh,
        out_type=jax.ShapeDtypeStruct((_L, _DP), jnp.int32),
        scratch_types=[
            pltpu.VMEM((_ROWS_W,), jnp.int32),
            pltpu.VMEM((_GCH, _DP), jnp.int32),
            pltpu.VMEM((_GCH, _DP), jnp.int32),
            pltpu.SemaphoreType.DMA,
            pltpu.SemaphoreType.DMA,
        ],
    )
    def k(x_hbm, st_hbm, xs_hbm, idx_v, rows_a, rows_b, sem_a, sem_b):
        wid = lax.axis_index("s") * 2 + lax.axis_index("c")
        base = wid * _ROWS_W
        pltpu.sync_copy(st_hbm.at[pl.ds(base, _ROWS_W)], idx_v)
        cp_a = pltpu.async_copy(
            x_hbm.at[idx_v.at[pl.ds(0, _GCH)]], rows_a, sem_a)
        cp_b = pltpu.async_copy(
            x_hbm.at[idx_v.at[pl.ds(_GCH, _GCH)]], rows_b, sem_b)
        cp_a.wait()
        pltpu.sync_copy(rows_a, xs_hbm.at[pl.ds(base, _GCH)])
        cp_b.wait()
        pltpu.sync_copy(rows_b, xs_hbm.at[pl.ds(base + _GCH, _GCH)])

    return k(xp, st_flat)


def _gmm_body(bexp_sref, xs_ref, gup_ref, down_ref, sw_ref, ys_ref):
    x = xs_ref[...]
    gu = lax.dot_general(x, gup_ref[0].astype(jnp.bfloat16),
                         (((1,), (1,)), ((), ())),
                         preferred_element_type=jnp.float32)
    g = gu[:, :_I]
    u = gu[:, _I:]
    h = (g * lax.logistic(g) * u).astype(jnp.bfloat16)
    y = lax.dot_general(h, down_ref[0].astype(jnp.bfloat16),
                        (((1,), (1,)), ((), ())),
                        preferred_element_type=jnp.float32)
    ys_ref[...] = y * sw_ref[...]


def _grouped_mm(bexp, xs, gate_up_proj, down_proj, sw_col):
    grid_spec = pltpu.PrefetchScalarGridSpec(
        num_scalar_prefetch=1,
        grid=(_NB,),
        in_specs=[
            pl.BlockSpec((_BLK, _D), lambda b, be: (b, 0)),
            pl.BlockSpec((1, 2 * _I, _D), lambda b, be: (be[b], 0, 0)),
            pl.BlockSpec((1, _D, _I), lambda b, be: (be[b], 0, 0)),
            pl.BlockSpec((_BLK, 1), lambda b, be: (b, 0)),
        ],
        out_specs=pl.BlockSpec((_BLK, _D), lambda b, be: (b, 0)),
    )
    return pl.pallas_call(
        _gmm_body,
        grid_spec=grid_spec,
        out_shape=jax.ShapeDtypeStruct((_L, _D), jnp.float32),
    )(bexp, xs, gate_up_proj, down_proj, sw_col)


def _shared_body(x_ref, gw_ref, uw_ref, dw_ref, sg_ref, out_ref):
    xf = x_ref[...]
    x = xf.astype(jnp.bfloat16)
    gs = lax.dot_general(x, gw_ref[...].astype(jnp.bfloat16),
                         (((1,), (1,)), ((), ())),
                         preferred_element_type=jnp.float32)
    us = lax.dot_general(x, uw_ref[...].astype(jnp.bfloat16),
                         (((1,), (1,)), ((), ())),
                         preferred_element_type=jnp.float32)
    hs = (gs * lax.logistic(gs) * us).astype(jnp.bfloat16)
    sh = lax.dot_general(hs, dw_ref[...].astype(jnp.bfloat16),
                         (((1,), (1,)), ((), ())),
                         preferred_element_type=jnp.float32)
    sgate = lax.logistic(
        lax.dot_general(xf, sg_ref[...], (((1,), (1,)), ((), ())),
                        preferred_element_type=jnp.float32))
    out_ref[...] = sgate * sh


def _shared_expert(x, gate_w, up_w, down_w, shared_gate_w):
    return pl.pallas_call(
        _shared_body,
        grid=(_T // _TB,),
        in_specs=[
            pl.BlockSpec((_TB, _D), lambda i: (i, 0)),
            pl.BlockSpec((_SI, _D), lambda i: (0, 0)),
            pl.BlockSpec((_SI, _D), lambda i: (0, 0)),
            pl.BlockSpec((_D, _SI), lambda i: (0, 0)),
            pl.BlockSpec((1, _D), lambda i: (0, 0)),
        ],
        out_specs=pl.BlockSpec((_TB, _D), lambda i: (i, 0)),
        out_shape=jax.ShapeDtypeStruct((_T, _D), jnp.float32),
    )(x, gate_w, up_w, down_w, shared_gate_w)


_TOK_W = _T // _NW
_CCH = 16


def _sc_combine(shared, ys, slot_flat):
    mesh = plsc.VectorSubcoreMesh(core_axis_name="c", subcore_axis_name="s")

    @functools.partial(
        pl.kernel, mesh=mesh,
        out_type=jax.ShapeDtypeStruct((_T, _D), jnp.float32),
        scratch_types=[
            pltpu.VMEM((_TOK_W,), jnp.int32),
            pltpu.VMEM((_TOK_W,), jnp.int32),
            pltpu.VMEM((_CCH, _D), jnp.float32),
            pltpu.VMEM((_CCH, _D), jnp.float32),
            pltpu.VMEM((_CCH, _D), jnp.float32),
            pltpu.SemaphoreType.DMA,
        ],
    )
    def k(shared_hbm, ys_hbm, slot_hbm, out_hbm, s0_v, s1_v, r0, r1, sh, sem):
        wid = lax.axis_index("s") * 2 + lax.axis_index("c")
        tbase = wid * _TOK_W
        pltpu.sync_copy(slot_hbm.at[pl.ds(tbase, _TOK_W)], s0_v)
        pltpu.sync_copy(slot_hbm.at[pl.ds(_T + tbase, _TOK_W)], s1_v)
        for c in range(_TOK_W // _CCH):
            cp0 = pltpu.async_copy(
                ys_hbm.at[s0_v.at[pl.ds(c * _CCH, _CCH)]], r0, sem)
            cp1 = pltpu.async_copy(
                ys_hbm.at[s1_v.at[pl.ds(c * _CCH, _CCH)]], r1, sem)
            pltpu.sync_copy(shared_hbm.at[pl.ds(tbase + c * _CCH, _CCH)], sh)
            cp0.wait()
            cp1.wait()
            for t in range(_CCH):
                def body(i, _):
                    sl = pl.ds(i * 16, 16)
                    sh[t, sl] = sh[t, sl] + r0[t, sl] + r1[t, sl]
                    return 0
                lax.fori_loop(0, _D // 16, body, 0)
            pltpu.sync_copy(sh, out_hbm.at[pl.ds(tbase + c * _CCH, _CCH)])

    return k(shared, ys, slot_flat)


@jax.jit
def kernel(hidden_states, router_weight, gate_up_proj, down_proj,
           gate_w, up_w, down_w, shared_gate_w):
    s, b, d = hidden_states.shape
    x = hidden_states.reshape(-1, d)

    probs, slot, st, sw, bexp = _dispatch(x, router_weight)
    xp = lax.bitcast_convert_type(
        x.astype(jnp.bfloat16).reshape(_T, _DP, 2), jnp.int32)
    xsp = _sc_gather(xp, st.reshape(_L))
    xs = lax.bitcast_convert_type(xsp, jnp.bfloat16).reshape(_L, _D)
    ys = _grouped_mm(bexp.reshape(64), xs, gate_up_proj, down_proj,
                     sw.reshape(_L, 1))
    shared = _shared_expert(x, gate_w, up_w, down_w, shared_gate_w)
    out = _sc_combine(shared, ys, slot.reshape(_P))

    return out.reshape(s, b, d), probs


# speedup vs baseline: 1.0345x; 1.0345x over previous
import functools

import jax
import jax.numpy as jnp
from jax import lax
from jax.experimental import pallas as pl
from jax.experimental.pallas import tpu as pltpu
from jax.experimental.pallas import tpu_sc as plsc

_T, _D = 2048, 1024
_E, _TOPK = 8, 2
_I = 512
_SI = 512
_P = _T * _TOPK
_BLK = 128
_NB = _P // _BLK + _E
_L = _NB * _BLK
_NW = 32
_TB = 256


def _dispatch_body(x_ref, wr_ref, probs_ref, slot_ref, st_ref, sw_ref,
                   bexp_ref, rank_ref, ep_ref, wp_ref):
    x = x_ref[...]
    logits = lax.dot_general(x, wr_ref[...], (((1,), (1,)), ((), ())),
                             preferred_element_type=jnp.float32)
    m = jnp.max(logits, axis=-1, keepdims=True)
    ex = jnp.exp(logits - m)
    probs = ex / jnp.sum(ex, axis=-1, keepdims=True)
    probs_ref[...] = probs

    iota_e = lax.broadcasted_iota(jnp.int32, (_T, _E), 1)
    i1 = jnp.argmax(probs, axis=-1)[:, None]
    oh1 = (iota_e == i1)
    m1 = jnp.max(probs, axis=-1, keepdims=True)
    masked = jnp.where(oh1, -jnp.inf, probs)
    i2 = jnp.argmax(masked, axis=-1)[:, None]
    m2 = jnp.max(masked, axis=-1, keepdims=True)
    denom = m1 + m2 + 1e-9

    ep_ref[0:_T, :] = i1
    ep_ref[_T:_P, :] = i2
    wp_ref[0:_T, :] = m1 / denom
    wp_ref[_T:_P, :] = m2 / denom
    ep = ep_ref[...]

    iota_e8 = lax.broadcasted_iota(jnp.int32, (1, _E), 1)
    tri = (lax.broadcasted_iota(jnp.int32, (_BLK, _BLK), 0)
           >= lax.broadcasted_iota(jnp.int32, (_BLK, _BLK), 1)
           ).astype(jnp.float32)

    def cum_body(c, carry):
        ep_c = ep_ref[pl.ds(c * _BLK, _BLK), :]
        a_c = (ep_c == iota_e8).astype(jnp.float32)
        c_c = lax.dot_general(tri, a_c, (((1,), (0,)), ((), ())),
                              preferred_element_type=jnp.float32) + carry
        rank_c = jnp.sum(c_c * a_c, axis=1, keepdims=True) - 1.0
        rank_ref[pl.ds(c * _BLK, _BLK), :] = rank_c
        return lax.slice(c_c, (_BLK - 1, 0), (_BLK, _E))

    counts = lax.fori_loop(0, _P // _BLK, cum_body,
                           jnp.zeros((1, _E), jnp.float32))

    pcnt = jnp.floor((counts + (_BLK - 1)) / _BLK) * _BLK
    below = (lax.broadcasted_iota(jnp.int32, (_E, _E), 0)
             < lax.broadcasted_iota(jnp.int32, (_E, _E), 1)).astype(jnp.float32)
    off = lax.dot_general(pcnt, below, (((1,), (0,)), ((), ())),
                          preferred_element_type=jnp.float32)
    bend = (off + pcnt) / _BLK
    iota_b = lax.broadcasted_iota(jnp.int32, (64, 1), 0).astype(jnp.float32)
    bexp = jnp.sum((bend <= iota_b).astype(jnp.int32), axis=1, keepdims=True)
    bexp_ref[...] = jnp.minimum(bexp, _E - 1)

    off_g = jnp.sum(jnp.where(ep == iota_e8, off, 0.0), axis=1, keepdims=True)
    slot = off_g + rank_ref[...]
    slot_ref[...] = slot.astype(jnp.int32)

    iota_j = lax.broadcasted_iota(jnp.int32, (_BLK, 1), 0)
    iota_l = lax.broadcasted_iota(jnp.int32, (1, _L), 1)

    def sc_body(c, carry):
        st_acc, sw_acc = carry
        slot_c = slot_ref[pl.ds(c * _BLK, _BLK), :]
        p_c = iota_j + c * _BLK
        tok_c = jnp.where(p_c >= _T, p_c - _T, p_c).astype(jnp.float32)
        wp_c = wp_ref[pl.ds(c * _BLK, _BLK), :]
        hit = (slot_c == iota_l)
        st_acc = st_acc + jnp.sum(jnp.where(hit, tok_c, 0.0), axis=0,
                                  keepdims=True)
        sw_acc = sw_acc + jnp.sum(jnp.where(hit, wp_c, 0.0), axis=0,
                                  keepdims=True)
        return st_acc, sw_acc

    st_acc, sw_acc = lax.fori_loop(
        0, _P // _BLK, sc_body,
        (jnp.zeros((1, _L), jnp.float32), jnp.zeros((1, _L), jnp.float32)))
    st_ref[...] = st_acc.astype(jnp.int32)
    sw_ref[...] = sw_acc


def _dispatch(x, router_weight):
    return pl.pallas_call(
        _dispatch_body,
        grid=(1,),
        in_specs=[
            pl.BlockSpec((_T, _D), lambda i: (0, 0)),
            pl.BlockSpec((_E, _D), lambda i: (0, 0)),
        ],
        out_specs=[
            pl.BlockSpec((_T, _E), lambda i: (0, 0)),
            pl.BlockSpec((_P, 1), lambda i: (0, 0)),
            pl.BlockSpec((1, _L), lambda i: (0, 0)),
            pl.BlockSpec((1, _L), lambda i: (0, 0)),
            pl.BlockSpec((64, 1), lambda i: (0, 0)),
        ],
        out_shape=[
            jax.ShapeDtypeStruct((_T, _E), jnp.float32),
            jax.ShapeDtypeStruct((_P, 1), jnp.int32),
            jax.ShapeDtypeStruct((1, _L), jnp.int32),
            jax.ShapeDtypeStruct((1, _L), jnp.float32),
            jax.ShapeDtypeStruct((64, 1), jnp.int32),
        ],
        scratch_shapes=[pltpu.VMEM((_P, 1), jnp.float32),
                        pltpu.VMEM((_P, 1), jnp.int32),
                        pltpu.VMEM((_P, 1), jnp.float32)],
    )(x, router_weight)


_ROWS_W = _L // _NW
_GCH = 80
_DP = _D // 2


def _sc_gather(xp, st_flat):
    mesh = plsc.VectorSubcoreMesh(core_axis_name="c", subcore_axis_name="s")

    @functools.partial(
        pl.kernel, mesh=mesh,
        out_type=jax.ShapeDtypeStruct((_L, _DP), jnp.int32),
        scratch_types=[
            pltpu.VMEM((_ROWS_W,), jnp.int32),
            pltpu.VMEM((_GCH, _DP), jnp.int32),
            pltpu.VMEM((_GCH, _DP), jnp.int32),
            pltpu.SemaphoreType.DMA,
            pltpu.SemaphoreType.DMA,
        ],
    )
    def k(x_hbm, st_hbm, xs_hbm, idx_v, rows_a, rows_b, sem_a, sem_b):
        wid = lax.axis_index("s") * 2 + lax.axis_index("c")
        base = wid * _ROWS_W
        pltpu.sync_copy(st_hbm.at[pl.ds(base, _ROWS_W)], idx_v)
        cp_a = pltpu.async_copy(
            x_hbm.at[idx_v.at[pl.ds(0, _GCH)]], rows_a, sem_a)
        cp_b = pltpu.async_copy(
            x_hbm.at[idx_v.at[pl.ds(_GCH, _GCH)]], rows_b, sem_b)
        cp_a.wait()
        pltpu.sync_copy(rows_a, xs_hbm.at[pl.ds(base, _GCH)])
        cp_b.wait()
        pltpu.sync_copy(rows_b, xs_hbm.at[pl.ds(base + _GCH, _GCH)])

    return k(xp, st_flat)


def _gmm_body(bexp_sref, xs_ref, gup_ref, down_ref, sw_ref, ys_ref,
              gup_bf, down_bf):
    b = pl.program_id(0)
    new_expert = (b == 0) | (bexp_sref[b] != bexp_sref[jnp.maximum(b - 1, 0)])

    @pl.when(new_expert)
    def _():
        gup_bf[...] = gup_ref[0].astype(jnp.bfloat16)
        down_bf[...] = down_ref[0].astype(jnp.bfloat16)

    x = xs_ref[...]
    gu = lax.dot_general(x, gup_bf[...], (((1,), (1,)), ((), ())),
                         preferred_element_type=jnp.float32)
    g = gu[:, :_I]
    u = gu[:, _I:]
    h = (g * lax.logistic(g) * u).astype(jnp.bfloat16)
    y = lax.dot_general(h, down_bf[...], (((1,), (1,)), ((), ())),
                        preferred_element_type=jnp.float32)
    ys_ref[...] = y * sw_ref[...]


def _grouped_mm(bexp, xs, gate_up_proj, down_proj, sw_col):
    grid_spec = pltpu.PrefetchScalarGridSpec(
        num_scalar_prefetch=1,
        grid=(_NB,),
        in_specs=[
            pl.BlockSpec((_BLK, _D), lambda b, be: (b, 0)),
            pl.BlockSpec((1, 2 * _I, _D), lambda b, be: (be[b], 0, 0)),
            pl.BlockSpec((1, _D, _I), lambda b, be: (be[b], 0, 0)),
            pl.BlockSpec((_BLK, 1), lambda b, be: (b, 0)),
        ],
        out_specs=pl.BlockSpec((_BLK, _D), lambda b, be: (b, 0)),
        scratch_shapes=[pltpu.VMEM((2 * _I, _D), jnp.bfloat16),
                        pltpu.VMEM((_D, _I), jnp.bfloat16)],
    )
    return pl.pallas_call(
        _gmm_body,
        grid_spec=grid_spec,
        out_shape=jax.ShapeDtypeStruct((_L, _D), jnp.float32),
    )(bexp, xs, gate_up_proj, down_proj, sw_col)


def _shared_body(x_ref, gw_ref, uw_ref, dw_ref, sg_ref, out_ref,
                 gw_bf, uw_bf, dw_bf):
    @pl.when(pl.program_id(0) == 0)
    def _():
        gw_bf[...] = gw_ref[...].astype(jnp.bfloat16)
        uw_bf[...] = uw_ref[...].astype(jnp.bfloat16)
        dw_bf[...] = dw_ref[...].astype(jnp.bfloat16)

    xf = x_ref[...]
    x = xf.astype(jnp.bfloat16)
    gs = lax.dot_general(x, gw_bf[...], (((1,), (1,)), ((), ())),
                         preferred_element_type=jnp.float32)
    us = lax.dot_general(x, uw_bf[...], (((1,), (1,)), ((), ())),
                         preferred_element_type=jnp.float32)
    hs = (gs * lax.logistic(gs) * us).astype(jnp.bfloat16)
    sh = lax.dot_general(hs, dw_bf[...], (((1,), (1,)), ((), ())),
                         preferred_element_type=jnp.float32)
    sgate = lax.logistic(
        lax.dot_general(xf, sg_ref[...], (((1,), (1,)), ((), ())),
                        preferred_element_type=jnp.float32))
    out_ref[...] = sgate * sh


def _shared_expert(x, gate_w, up_w, down_w, shared_gate_w):
    return pl.pallas_call(
        _shared_body,
        grid=(_T // _TB,),
        in_specs=[
            pl.BlockSpec((_TB, _D), lambda i: (i, 0)),
            pl.BlockSpec((_SI, _D), lambda i: (0, 0)),
            pl.BlockSpec((_SI, _D), lambda i: (0, 0)),
            pl.BlockSpec((_D, _SI), lambda i: (0, 0)),
            pl.BlockSpec((1, _D), lambda i: (0, 0)),
        ],
        out_specs=pl.BlockSpec((_TB, _D), lambda i: (i, 0)),
        out_shape=jax.ShapeDtypeStruct((_T, _D), jnp.float32),
        scratch_shapes=[pltpu.VMEM((_SI, _D), jnp.bfloat16),
                        pltpu.VMEM((_SI, _D), jnp.bfloat16),
                        pltpu.VMEM((_D, _SI), jnp.bfloat16)],
    )(x, gate_w, up_w, down_w, shared_gate_w)


_TOK_W = _T // _NW


def _sc_combine(shp, ysp, slot_flat):
    mesh = plsc.VectorSubcoreMesh(core_axis_name="c", subcore_axis_name="s")

    @functools.partial(
        pl.kernel, mesh=mesh,
        out_type=jax.ShapeDtypeStruct((_T, _D), jnp.float32),
        scratch_types=[
            pltpu.VMEM((_TOK_W,), jnp.int32),
            pltpu.VMEM((_TOK_W,), jnp.int32),
            pltpu.VMEM((_TOK_W // 2, _D), jnp.float32),
            pltpu.VMEM((_TOK_W // 2, _D), jnp.float32),
            pltpu.VMEM((_TOK_W // 2, _D), jnp.float32),
            pltpu.SemaphoreType.DMA,
            pltpu.SemaphoreType.DMA,
            pltpu.SemaphoreType.DMA,
        ],
    )
    def k(sh_hbm, ys_hbm, slot_hbm, out_hbm, s0_v, s1_v, r0, r1, sh,
          sem0, sem1, sem2):
        wid = lax.axis_index("s") * 2 + lax.axis_index("c")
        tbase = wid * _TOK_W
        half = _TOK_W // 2
        pltpu.sync_copy(slot_hbm.at[pl.ds(tbase, _TOK_W)], s0_v)
        pltpu.sync_copy(slot_hbm.at[pl.ds(_T + tbase, _TOK_W)], s1_v)
        for c in range(2):
            cp0 = pltpu.async_copy(
                ys_hbm.at[s0_v.at[pl.ds(c * half, half)]], r0, sem0)
            cp1 = pltpu.async_copy(
                ys_hbm.at[s1_v.at[pl.ds(c * half, half)]], r1, sem1)
            cp2 = pltpu.async_copy(
                sh_hbm.at[pl.ds(tbase + c * half, half)], sh, sem2)
            cp0.wait()
            cp1.wait()
            cp2.wait()
            for t in range(half):
                def body(i, _):
                    for u in range(8):
                        sl = pl.ds((i * 8 + u) * 16, 16)
                        sh[t, sl] = sh[t, sl] + r0[t, sl] + r1[t, sl]
                    return 0
                lax.fori_loop(0, _D // 128, body, 0)
            pltpu.sync_copy(sh, out_hbm.at[pl.ds(tbase + c * half, half)])

    return k(shp, ysp, slot_flat)


@jax.jit
def kernel(hidden_states, router_weight, gate_up_proj, down_proj,
           gate_w, up_w, down_w, shared_gate_w):
    s, b, d = hidden_states.shape
    x = hidden_states.reshape(-1, d)

    probs, slot, st, sw, bexp = _dispatch(x, router_weight)
    xp = lax.bitcast_convert_type(
        x.astype(jnp.bfloat16).reshape(_T, _DP, 2), jnp.int32)
    xsp = _sc_gather(xp, st.reshape(_L))
    xs = lax.bitcast_convert_type(xsp, jnp.bfloat16).reshape(_L, _D)
    ys = _grouped_mm(bexp.reshape(64), xs, gate_up_proj, down_proj,
                     sw.reshape(_L, 1))
    shared = _shared_expert(x, gate_w, up_w, down_w, shared_gate_w)
    out = _sc_combine(shared, ys, slot.reshape(_P))

    return out.astype(jnp.float32).reshape(s, b, d), probs


# speedup vs baseline: 1.6084x; 1.5548x over previous
import functools

import jax
import jax.numpy as jnp
from jax import lax
from jax.experimental import pallas as pl
from jax.experimental.pallas import tpu as pltpu
from jax.experimental.pallas import tpu_sc as plsc

_T, _D = 2048, 1024
_E, _TOPK = 8, 2
_I = 512
_SI = 512
_P = _T * _TOPK
_BLK = 128
_NB = _P // _BLK + _E
_L = _NB * _BLK
_NW = 32
_TB = 256


def _dispatch_body(x_ref, wr_ref, probs_ref, slot_ref, st_ref, sw_ref,
                   bexp_ref, rank_ref, ep_ref, wp_ref):
    x = x_ref[...]
    logits = lax.dot_general(x, wr_ref[...], (((1,), (1,)), ((), ())),
                             preferred_element_type=jnp.float32)
    m = jnp.max(logits, axis=-1, keepdims=True)
    ex = jnp.exp(logits - m)
    probs = ex / jnp.sum(ex, axis=-1, keepdims=True)
    probs_ref[...] = probs

    iota_e = lax.broadcasted_iota(jnp.int32, (_T, _E), 1)
    i1 = jnp.argmax(probs, axis=-1)[:, None]
    oh1 = (iota_e == i1)
    m1 = jnp.max(probs, axis=-1, keepdims=True)
    masked = jnp.where(oh1, -jnp.inf, probs)
    i2 = jnp.argmax(masked, axis=-1)[:, None]
    m2 = jnp.max(masked, axis=-1, keepdims=True)
    denom = m1 + m2 + 1e-9

    ep_ref[0:_T, :] = i1
    ep_ref[_T:_P, :] = i2
    wp_ref[0:_T, :] = m1 / denom
    wp_ref[_T:_P, :] = m2 / denom
    ep = ep_ref[...]

    iota_e8 = lax.broadcasted_iota(jnp.int32, (1, _E), 1)
    tri = (lax.broadcasted_iota(jnp.int32, (_BLK, _BLK), 0)
           >= lax.broadcasted_iota(jnp.int32, (_BLK, _BLK), 1)
           ).astype(jnp.float32)

    def cum_body(c, carry):
        ep_c = ep_ref[pl.ds(c * _BLK, _BLK), :]
        a_c = (ep_c == iota_e8).astype(jnp.float32)
        c_c = lax.dot_general(tri, a_c, (((1,), (0,)), ((), ())),
                              preferred_element_type=jnp.float32) + carry
        rank_c = jnp.sum(c_c * a_c, axis=1, keepdims=True) - 1.0
        rank_ref[pl.ds(c * _BLK, _BLK), :] = rank_c
        return lax.slice(c_c, (_BLK - 1, 0), (_BLK, _E))

    counts = lax.fori_loop(0, _P // _BLK, cum_body,
                           jnp.zeros((1, _E), jnp.float32))

    pcnt = jnp.floor((counts + (_BLK - 1)) / _BLK) * _BLK
    below = (lax.broadcasted_iota(jnp.int32, (_E, _E), 0)
             < lax.broadcasted_iota(jnp.int32, (_E, _E), 1)).astype(jnp.float32)
    off = lax.dot_general(pcnt, below, (((1,), (0,)), ((), ())),
                          preferred_element_type=jnp.float32)
    bend = (off + pcnt) / _BLK
    iota_b = lax.broadcasted_iota(jnp.int32, (64, 1), 0).astype(jnp.float32)
    bexp = jnp.sum((bend <= iota_b).astype(jnp.int32), axis=1, keepdims=True)
    bexp_ref[...] = jnp.minimum(bexp, _E - 1)

    off_g = jnp.sum(jnp.where(ep == iota_e8, off, 0.0), axis=1, keepdims=True)
    slot = off_g + rank_ref[...]
    slot_ref[...] = slot.astype(jnp.int32)

    iota_j = lax.broadcasted_iota(jnp.int32, (_BLK, 1), 0)
    iota_l = lax.broadcasted_iota(jnp.int32, (1, _L), 1)

    def sc_body(c, carry):
        st_acc, sw_acc = carry
        slot_c = slot_ref[pl.ds(c * _BLK, _BLK), :]
        p_c = iota_j + c * _BLK
        tok_c = jnp.where(p_c >= _T, p_c - _T, p_c).astype(jnp.float32)
        wp_c = wp_ref[pl.ds(c * _BLK, _BLK), :]
        hit = (slot_c == iota_l)
        st_acc = st_acc + jnp.sum(jnp.where(hit, tok_c, 0.0), axis=0,
                                  keepdims=True)
        sw_acc = sw_acc + jnp.sum(jnp.where(hit, wp_c, 0.0), axis=0,
                                  keepdims=True)
        return st_acc, sw_acc

    st_acc, sw_acc = lax.fori_loop(
        0, _P // _BLK, sc_body,
        (jnp.zeros((1, _L), jnp.float32), jnp.zeros((1, _L), jnp.float32)))
    st_ref[...] = st_acc.astype(jnp.int32)
    sw_ref[...] = sw_acc


def _dispatch(x, router_weight):
    return pl.pallas_call(
        _dispatch_body,
        grid=(1,),
        in_specs=[
            pl.BlockSpec((_T, _D), lambda i: (0, 0)),
            pl.BlockSpec((_E, _D), lambda i: (0, 0)),
        ],
        out_specs=[
            pl.BlockSpec((_T, _E), lambda i: (0, 0)),
            pl.BlockSpec((_P, 1), lambda i: (0, 0)),
            pl.BlockSpec((1, _L), lambda i: (0, 0)),
            pl.BlockSpec((1, _L), lambda i: (0, 0)),
            pl.BlockSpec((64, 1), lambda i: (0, 0)),
        ],
        out_shape=[
            jax.ShapeDtypeStruct((_T, _E), jnp.float32),
            jax.ShapeDtypeStruct((_P, 1), jnp.int32),
            jax.ShapeDtypeStruct((1, _L), jnp.int32),
            jax.ShapeDtypeStruct((1, _L), jnp.float32),
            jax.ShapeDtypeStruct((64, 1), jnp.int32),
        ],
        scratch_shapes=[pltpu.VMEM((_P, 1), jnp.float32),
                        pltpu.VMEM((_P, 1), jnp.int32),
                        pltpu.VMEM((_P, 1), jnp.float32)],
    )(x, router_weight)


_ROWS_W = _L // _NW
_GCH = 32


def _sc_gather(x, st_flat):
    mesh = plsc.VectorSubcoreMesh(core_axis_name="c", subcore_axis_name="s")

    @functools.partial(
        pl.kernel, mesh=mesh,
        out_type=jax.ShapeDtypeStruct((_L, _D), jnp.float32),
        scratch_types=[
            pltpu.VMEM((_ROWS_W,), jnp.int32),
            pltpu.VMEM((_GCH, _D), jnp.float32),
            pltpu.VMEM((_GCH, _D), jnp.float32),
            pltpu.SemaphoreType.DMA,
            pltpu.SemaphoreType.DMA,
        ],
    )
    def k(x_hbm, st_hbm, xs_hbm, idx_v, rows_a, rows_b, sem_a, sem_b):
        wid = lax.axis_index("s") * 2 + lax.axis_index("c")
        base = wid * _ROWS_W
        pltpu.sync_copy(st_hbm.at[pl.ds(base, _ROWS_W)], idx_v)
        bufs = ((rows_a, sem_a), (rows_b, sem_b))
        nch = _ROWS_W // _GCH
        cps = [None] * nch
        for c in range(nch):
            rows, sem = bufs[c % 2]
            cps[c] = pltpu.async_copy(
                x_hbm.at[idx_v.at[pl.ds(c * _GCH, _GCH)]], rows, sem)
            if c >= 1:
                pr, _ = bufs[(c - 1) % 2]
                cps[c - 1].wait()
                pltpu.sync_copy(
                    pr, xs_hbm.at[pl.ds(base + (c - 1) * _GCH, _GCH)])
        rows, _ = bufs[(nch - 1) % 2]
        cps[nch - 1].wait()
        pltpu.sync_copy(rows, xs_hbm.at[pl.ds(base + (nch - 1) * _GCH, _GCH)])

    return k(x, st_flat)


def _gmm_body(bexp_sref, xs_ref, gup_ref, down_ref, sw_ref, ys_ref,
              gup_bf, down_bf):
    b = pl.program_id(0)
    new_expert = (b == 0) | (bexp_sref[b] != bexp_sref[jnp.maximum(b - 1, 0)])

    @pl.when(new_expert)
    def _():
        gup_bf[...] = gup_ref[0].astype(jnp.bfloat16)
        down_bf[...] = down_ref[0].astype(jnp.bfloat16)

    x = xs_ref[...].astype(jnp.bfloat16)
    gu = lax.dot_general(x, gup_bf[...], (((1,), (1,)), ((), ())),
                         preferred_element_type=jnp.float32)
    g = gu[:, :_I]
    u = gu[:, _I:]
    h = (g * lax.logistic(g) * u).astype(jnp.bfloat16)
    y = lax.dot_general(h, down_bf[...], (((1,), (1,)), ((), ())),
                        preferred_element_type=jnp.float32)
    ys_ref[...] = y * sw_ref[...]


def _grouped_mm(bexp, xs, gate_up_proj, down_proj, sw_col):
    grid_spec = pltpu.PrefetchScalarGridSpec(
        num_scalar_prefetch=1,
        grid=(_NB,),
        in_specs=[
            pl.BlockSpec((_BLK, _D), lambda b, be: (b, 0)),
            pl.BlockSpec((1, 2 * _I, _D), lambda b, be: (be[b], 0, 0)),
            pl.BlockSpec((1, _D, _I), lambda b, be: (be[b], 0, 0)),
            pl.BlockSpec((_BLK, 1), lambda b, be: (b, 0)),
        ],
        out_specs=pl.BlockSpec((_BLK, _D), lambda b, be: (b, 0)),
        scratch_shapes=[pltpu.VMEM((2 * _I, _D), jnp.bfloat16),
                        pltpu.VMEM((_D, _I), jnp.bfloat16)],
    )
    return pl.pallas_call(
        _gmm_body,
        grid_spec=grid_spec,
        out_shape=jax.ShapeDtypeStruct((_L, _D), jnp.float32),
    )(bexp, xs, gate_up_proj, down_proj, sw_col)


def _shared_body(x_ref, gw_ref, uw_ref, dw_ref, sg_ref, out_ref,
                 gw_bf, uw_bf, dw_bf):
    @pl.when(pl.program_id(0) == 0)
    def _():
        gw_bf[...] = gw_ref[...].astype(jnp.bfloat16)
        uw_bf[...] = uw_ref[...].astype(jnp.bfloat16)
        dw_bf[...] = dw_ref[...].astype(jnp.bfloat16)

    xf = x_ref[...]
    x = xf.astype(jnp.bfloat16)
    gs = lax.dot_general(x, gw_bf[...], (((1,), (1,)), ((), ())),
                         preferred_element_type=jnp.float32)
    us = lax.dot_general(x, uw_bf[...], (((1,), (1,)), ((), ())),
                         preferred_element_type=jnp.float32)
    hs = (gs * lax.logistic(gs) * us).astype(jnp.bfloat16)
    sh = lax.dot_general(hs, dw_bf[...], (((1,), (1,)), ((), ())),
                         preferred_element_type=jnp.float32)
    sgate = lax.logistic(
        lax.dot_general(xf, sg_ref[...], (((1,), (1,)), ((), ())),
                        preferred_element_type=jnp.float32))
    out_ref[...] = sgate * sh


def _shared_expert(x, gate_w, up_w, down_w, shared_gate_w):
    return pl.pallas_call(
        _shared_body,
        grid=(_T // _TB,),
        in_specs=[
            pl.BlockSpec((_TB, _D), lambda i: (i, 0)),
            pl.BlockSpec((_SI, _D), lambda i: (0, 0)),
            pl.BlockSpec((_SI, _D), lambda i: (0, 0)),
            pl.BlockSpec((_D, _SI), lambda i: (0, 0)),
            pl.BlockSpec((1, _D), lambda i: (0, 0)),
        ],
        out_specs=pl.BlockSpec((_TB, _D), lambda i: (i, 0)),
        out_shape=jax.ShapeDtypeStruct((_T, _D), jnp.float32),
        scratch_shapes=[pltpu.VMEM((_SI, _D), jnp.bfloat16),
                        pltpu.VMEM((_SI, _D), jnp.bfloat16),
                        pltpu.VMEM((_D, _SI), jnp.bfloat16)],
    )(x, gate_w, up_w, down_w, shared_gate_w)


_TOK_W = _T // _NW


def _sc_combine(shp, ysp, slot_flat):
    mesh = plsc.VectorSubcoreMesh(core_axis_name="c", subcore_axis_name="s")

    @functools.partial(
        pl.kernel, mesh=mesh,
        out_type=jax.ShapeDtypeStruct((_T, _D), jnp.float32),
        scratch_types=[
            pltpu.VMEM((_TOK_W,), jnp.int32),
            pltpu.VMEM((_TOK_W,), jnp.int32),
            pltpu.VMEM((_TOK_W // 2, _D), jnp.float32),
            pltpu.VMEM((_TOK_W // 2, _D), jnp.float32),
            pltpu.VMEM((_TOK_W // 2, _D), jnp.float32),
            pltpu.SemaphoreType.DMA,
            pltpu.SemaphoreType.DMA,
            pltpu.SemaphoreType.DMA,
        ],
    )
    def k(sh_hbm, ys_hbm, slot_hbm, out_hbm, s0_v, s1_v, r0, r1, sh,
          sem0, sem1, sem2):
        wid = lax.axis_index("s") * 2 + lax.axis_index("c")
        tbase = wid * _TOK_W
        half = _TOK_W // 2
        pltpu.sync_copy(slot_hbm.at[pl.ds(tbase, _TOK_W)], s0_v)
        pltpu.sync_copy(slot_hbm.at[pl.ds(_T + tbase, _TOK_W)], s1_v)
        for c in range(2):
            cp0 = pltpu.async_copy(
                ys_hbm.at[s0_v.at[pl.ds(c * half, half)]], r0, sem0)
            cp1 = pltpu.async_copy(
                ys_hbm.at[s1_v.at[pl.ds(c * half, half)]], r1, sem1)
            cp2 = pltpu.async_copy(
                sh_hbm.at[pl.ds(tbase + c * half, half)], sh, sem2)
            cp0.wait()
            cp1.wait()
            cp2.wait()
            for t in range(half):
                def body(i, _):
                    for u in range(8):
                        sl = pl.ds((i * 8 + u) * 16, 16)
                        sh[t, sl] = sh[t, sl] + r0[t, sl] + r1[t, sl]
                    return 0
                lax.fori_loop(0, _D // 128, body, 0)
            pltpu.sync_copy(sh, out_hbm.at[pl.ds(tbase + c * half, half)])

    return k(shp, ysp, slot_flat)


@jax.jit
def kernel(hidden_states, router_weight, gate_up_proj, down_proj,
           gate_w, up_w, down_w, shared_gate_w):
    s, b, d = hidden_states.shape
    x = hidden_states.reshape(-1, d)

    probs, slot, st, sw, bexp = _dispatch(x, router_weight)
    xs = _sc_gather(x, st.reshape(_L))
    ys = _grouped_mm(bexp.reshape(64), xs, gate_up_proj, down_proj,
                     sw.reshape(_L, 1))
    shared = _shared_expert(x, gate_w, up_w, down_w, shared_gate_w)
    out = _sc_combine(shared, ys, slot.reshape(_P))

    return out.astype(jnp.float32).reshape(s, b, d), probs


# speedup vs baseline: 1.6094x; 1.0006x over previous
import functools

import jax
import jax.numpy as jnp
from jax import lax
from jax.experimental import pallas as pl
from jax.experimental.pallas import tpu as pltpu
from jax.experimental.pallas import tpu_sc as plsc

_T, _D = 2048, 1024
_E, _TOPK = 8, 2
_I = 512
_SI = 512
_P = _T * _TOPK
_BLK = 128
_NB = _P // _BLK + _E
_L = _NB * _BLK
_NW = 32
_TB = 256


def _dispatch_body(x_ref, wr_ref, probs_ref, slot_ref, st_ref, sw_ref,
                   bexp_ref, rank_ref, ep_ref, wp_ref):
    x = x_ref[...]
    logits = lax.dot_general(x, wr_ref[...], (((1,), (1,)), ((), ())),
                             preferred_element_type=jnp.float32)
    m = jnp.max(logits, axis=-1, keepdims=True)
    ex = jnp.exp(logits - m)
    probs = ex / jnp.sum(ex, axis=-1, keepdims=True)
    probs_ref[...] = probs

    iota_e = lax.broadcasted_iota(jnp.int32, (_T, _E), 1)
    i1 = jnp.argmax(probs, axis=-1)[:, None]
    oh1 = (iota_e == i1)
    m1 = jnp.max(probs, axis=-1, keepdims=True)
    masked = jnp.where(oh1, -jnp.inf, probs)
    i2 = jnp.argmax(masked, axis=-1)[:, None]
    m2 = jnp.max(masked, axis=-1, keepdims=True)
    denom = m1 + m2 + 1e-9

    ep_ref[0:_T, :] = i1
    ep_ref[_T:_P, :] = i2
    wp_ref[0:_T, :] = m1 / denom
    wp_ref[_T:_P, :] = m2 / denom
    ep = ep_ref[...]

    iota_e8 = lax.broadcasted_iota(jnp.int32, (1, _E), 1)
    tri = (lax.broadcasted_iota(jnp.int32, (_BLK, _BLK), 0)
           >= lax.broadcasted_iota(jnp.int32, (_BLK, _BLK), 1)
           ).astype(jnp.float32)

    def cum_body(c, carry):
        ep_c = ep_ref[pl.ds(c * _BLK, _BLK), :]
        a_c = (ep_c == iota_e8).astype(jnp.float32)
        c_c = lax.dot_general(tri, a_c, (((1,), (0,)), ((), ())),
                              preferred_element_type=jnp.float32) + carry
        rank_c = jnp.sum(c_c * a_c, axis=1, keepdims=True) - 1.0
        rank_ref[pl.ds(c * _BLK, _BLK), :] = rank_c
        return lax.slice(c_c, (_BLK - 1, 0), (_BLK, _E))

    counts = lax.fori_loop(0, _P // _BLK, cum_body,
                           jnp.zeros((1, _E), jnp.float32))

    pcnt = jnp.floor((counts + (_BLK - 1)) / _BLK) * _BLK
    below = (lax.broadcasted_iota(jnp.int32, (_E, _E), 0)
             < lax.broadcasted_iota(jnp.int32, (_E, _E), 1)).astype(jnp.float32)
    off = lax.dot_general(pcnt, below, (((1,), (0,)), ((), ())),
                          preferred_element_type=jnp.float32)
    bend = (off + pcnt) / _BLK
    iota_b = lax.broadcasted_iota(jnp.int32, (64, 1), 0).astype(jnp.float32)
    bexp = jnp.sum((bend <= iota_b).astype(jnp.int32), axis=1, keepdims=True)
    bexp_ref[...] = jnp.minimum(bexp, _E - 1)

    off_g = jnp.sum(jnp.where(ep == iota_e8, off, 0.0), axis=1, keepdims=True)
    slot = off_g + rank_ref[...]
    slot_ref[...] = slot.astype(jnp.int32)

    iota_j = lax.broadcasted_iota(jnp.int32, (_BLK, 1), 0)
    iota_l = lax.broadcasted_iota(jnp.int32, (1, _L), 1)

    def sc_body(c, carry):
        st_acc, sw_acc = carry
        slot_c = slot_ref[pl.ds(c * _BLK, _BLK), :]
        p_c = iota_j + c * _BLK
        tok_c = jnp.where(p_c >= _T, p_c - _T, p_c).astype(jnp.float32)
        wp_c = wp_ref[pl.ds(c * _BLK, _BLK), :]
        hit = (slot_c == iota_l)
        st_acc = st_acc + jnp.sum(jnp.where(hit, tok_c, 0.0), axis=0,
                                  keepdims=True)
        sw_acc = sw_acc + jnp.sum(jnp.where(hit, wp_c, 0.0), axis=0,
                                  keepdims=True)
        return st_acc, sw_acc

    st_acc, sw_acc = lax.fori_loop(
        0, _P // _BLK, sc_body,
        (jnp.zeros((1, _L), jnp.float32), jnp.zeros((1, _L), jnp.float32)))
    st_ref[...] = st_acc.astype(jnp.int32)
    sw_ref[...] = sw_acc


def _dispatch(x, router_weight):
    return pl.pallas_call(
        _dispatch_body,
        grid=(1,),
        in_specs=[
            pl.BlockSpec((_T, _D), lambda i: (0, 0)),
            pl.BlockSpec((_E, _D), lambda i: (0, 0)),
        ],
        out_specs=[
            pl.BlockSpec((_T, _E), lambda i: (0, 0)),
            pl.BlockSpec((_P, 1), lambda i: (0, 0)),
            pl.BlockSpec((1, _L), lambda i: (0, 0)),
            pl.BlockSpec((1, _L), lambda i: (0, 0)),
            pl.BlockSpec((64, 1), lambda i: (0, 0)),
        ],
        out_shape=[
            jax.ShapeDtypeStruct((_T, _E), jnp.float32),
            jax.ShapeDtypeStruct((_P, 1), jnp.int32),
            jax.ShapeDtypeStruct((1, _L), jnp.int32),
            jax.ShapeDtypeStruct((1, _L), jnp.float32),
            jax.ShapeDtypeStruct((64, 1), jnp.int32),
        ],
        scratch_shapes=[pltpu.VMEM((_P, 1), jnp.float32),
                        pltpu.VMEM((_P, 1), jnp.int32),
                        pltpu.VMEM((_P, 1), jnp.float32)],
    )(x, router_weight)


_ROWS_W = _L // _NW
_GCH = 32


def _sc_gather(x, st_flat):
    mesh = plsc.VectorSubcoreMesh(core_axis_name="c", subcore_axis_name="s")

    @functools.partial(
        pl.kernel, mesh=mesh,
        out_type=jax.ShapeDtypeStruct((_L, _D), jnp.float32),
        scratch_types=[
            pltpu.VMEM((_ROWS_W,), jnp.int32),
            pltpu.VMEM((_GCH, _D), jnp.float32),
            pltpu.VMEM((_GCH, _D), jnp.float32),
            pltpu.VMEM((_GCH, _D), jnp.float32),
            pltpu.SemaphoreType.DMA,
            pltpu.SemaphoreType.DMA,
            pltpu.SemaphoreType.DMA,
            pltpu.SemaphoreType.DMA,
            pltpu.SemaphoreType.DMA,
            pltpu.SemaphoreType.DMA,
        ],
    )
    def k(x_hbm, st_hbm, xs_hbm, idx_v, rows_a, rows_b, rows_c,
          ga, gb, gc, sa, sb, sc_):
        wid = lax.axis_index("s") * 2 + lax.axis_index("c")
        base = wid * _ROWS_W
        pltpu.sync_copy(st_hbm.at[pl.ds(base, _ROWS_W)], idx_v)
        bufs = ((rows_a, ga, sa), (rows_b, gb, sb), (rows_c, gc, sc_))
        nch = _ROWS_W // _GCH
        g_cps = [None] * nch
        s_cps = [None] * nch
        for c in range(nch):
            rows, gsem, ssem = bufs[c % 3]
            if c >= 3:
                s_cps[c - 3].wait()
            g_cps[c] = pltpu.async_copy(
                x_hbm.at[idx_v.at[pl.ds(c * _GCH, _GCH)]], rows, gsem)
            if c >= 1:
                pr, _, pssem = bufs[(c - 1) % 3]
                g_cps[c - 1].wait()
                s_cps[c - 1] = pltpu.async_copy(
                    pr, xs_hbm.at[pl.ds(base + (c - 1) * _GCH, _GCH)], pssem)
        rows, _, ssem = bufs[(nch - 1) % 3]
        g_cps[nch - 1].wait()
        s_cps[nch - 1] = pltpu.async_copy(
            rows, xs_hbm.at[pl.ds(base + (nch - 1) * _GCH, _GCH)], ssem)
        for c in range(max(nch - 3, 0), nch):
            if s_cps[c] is not None:
                s_cps[c].wait()

    return k(x, st_flat)


def _gmm_body(bexp_sref, xs_ref, gup_ref, down_ref, sw_ref, ys_ref,
              gup_bf, down_bf):
    b = pl.program_id(0)
    new_expert = (b == 0) | (bexp_sref[b] != bexp_sref[jnp.maximum(b - 1, 0)])

    @pl.when(new_expert)
    def _():
        gup_bf[...] = gup_ref[0].astype(jnp.bfloat16)
        down_bf[...] = down_ref[0].astype(jnp.bfloat16)

    x = xs_ref[...].astype(jnp.bfloat16)
    gu = lax.dot_general(x, gup_bf[...], (((1,), (1,)), ((), ())),
                         preferred_element_type=jnp.float32)
    g = gu[:, :_I]
    u = gu[:, _I:]
    h = (g * lax.logistic(g) * u).astype(jnp.bfloat16)
    y = lax.dot_general(h, down_bf[...], (((1,), (1,)), ((), ())),
                        preferred_element_type=jnp.float32)
    ys_ref[...] = y * sw_ref[...]


def _grouped_mm(bexp, xs, gate_up_proj, down_proj, sw_col):
    grid_spec = pltpu.PrefetchScalarGridSpec(
        num_scalar_prefetch=1,
        grid=(_NB,),
        in_specs=[
            pl.BlockSpec((_BLK, _D), lambda b, be: (b, 0)),
            pl.BlockSpec((1, 2 * _I, _D), lambda b, be: (be[b], 0, 0)),
            pl.BlockSpec((1, _D, _I), lambda b, be: (be[b], 0, 0)),
            pl.BlockSpec((_BLK, 1), lambda b, be: (b, 0)),
        ],
        out_specs=pl.BlockSpec((_BLK, _D), lambda b, be: (b, 0)),
        scratch_shapes=[pltpu.VMEM((2 * _I, _D), jnp.bfloat16),
                        pltpu.VMEM((_D, _I), jnp.bfloat16)],
    )
    return pl.pallas_call(
        _gmm_body,
        grid_spec=grid_spec,
        out_shape=jax.ShapeDtypeStruct((_L, _D), jnp.float32),
    )(bexp, xs, gate_up_proj, down_proj, sw_col)


def _shared_body(x_ref, gw_ref, uw_ref, dw_ref, sg_ref, out_ref,
                 gw_bf, uw_bf, dw_bf):
    @pl.when(pl.program_id(0) == 0)
    def _():
        gw_bf[...] = gw_ref[...].astype(jnp.bfloat16)
        uw_bf[...] = uw_ref[...].astype(jnp.bfloat16)
        dw_bf[...] = dw_ref[...].astype(jnp.bfloat16)

    xf = x_ref[...]
    x = xf.astype(jnp.bfloat16)
    gs = lax.dot_general(x, gw_bf[...], (((1,), (1,)), ((), ())),
                         preferred_element_type=jnp.float32)
    us = lax.dot_general(x, uw_bf[...], (((1,), (1,)), ((), ())),
                         preferred_element_type=jnp.float32)
    hs = (gs * lax.logistic(gs) * us).astype(jnp.bfloat16)
    sh = lax.dot_general(hs, dw_bf[...], (((1,), (1,)), ((), ())),
                         preferred_element_type=jnp.float32)
    sgate = lax.logistic(
        lax.dot_general(xf, sg_ref[...], (((1,), (1,)), ((), ())),
                        preferred_element_type=jnp.float32))
    out_ref[...] = sgate * sh


def _shared_expert(x, gate_w, up_w, down_w, shared_gate_w):
    return pl.pallas_call(
        _shared_body,
        grid=(_T // _TB,),
        in_specs=[
            pl.BlockSpec((_TB, _D), lambda i: (i, 0)),
            pl.BlockSpec((_SI, _D), lambda i: (0, 0)),
            pl.BlockSpec((_SI, _D), lambda i: (0, 0)),
            pl.BlockSpec((_D, _SI), lambda i: (0, 0)),
            pl.BlockSpec((1, _D), lambda i: (0, 0)),
        ],
        out_specs=pl.BlockSpec((_TB, _D), lambda i: (i, 0)),
        out_shape=jax.ShapeDtypeStruct((_T, _D), jnp.float32),
        scratch_shapes=[pltpu.VMEM((_SI, _D), jnp.bfloat16),
                        pltpu.VMEM((_SI, _D), jnp.bfloat16),
                        pltpu.VMEM((_D, _SI), jnp.bfloat16)],
    )(x, gate_w, up_w, down_w, shared_gate_w)


_TOK_W = _T // _NW


def _sc_combine(shp, ysp, slot_flat):
    mesh = plsc.VectorSubcoreMesh(core_axis_name="c", subcore_axis_name="s")

    @functools.partial(
        pl.kernel, mesh=mesh,
        out_type=jax.ShapeDtypeStruct((_T, _D), jnp.float32),
        scratch_types=[
            pltpu.VMEM((_TOK_W,), jnp.int32),
            pltpu.VMEM((_TOK_W,), jnp.int32),
            pltpu.VMEM((_TOK_W // 2, _D), jnp.float32),
            pltpu.VMEM((_TOK_W // 2, _D), jnp.float32),
            pltpu.VMEM((_TOK_W // 2, _D), jnp.float32),
            pltpu.SemaphoreType.DMA,
            pltpu.SemaphoreType.DMA,
            pltpu.SemaphoreType.DMA,
        ],
    )
    def k(sh_hbm, ys_hbm, slot_hbm, out_hbm, s0_v, s1_v, r0, r1, sh,
          sem0, sem1, sem2):
        wid = lax.axis_index("s") * 2 + lax.axis_index("c")
        tbase = wid * _TOK_W
        half = _TOK_W // 2
        pltpu.sync_copy(slot_hbm.at[pl.ds(tbase, _TOK_W)], s0_v)
        pltpu.sync_copy(slot_hbm.at[pl.ds(_T + tbase, _TOK_W)], s1_v)
        for c in range(2):
            cp0 = pltpu.async_copy(
                ys_hbm.at[s0_v.at[pl.ds(c * half, half)]], r0, sem0)
            cp1 = pltpu.async_copy(
                ys_hbm.at[s1_v.at[pl.ds(c * half, half)]], r1, sem1)
            cp2 = pltpu.async_copy(
                sh_hbm.at[pl.ds(tbase + c * half, half)], sh, sem2)
            cp0.wait()
            cp1.wait()
            cp2.wait()
            for t in range(half):
                def body(i, _):
                    for u in range(8):
                        sl = pl.ds((i * 8 + u) * 16, 16)
                        sh[t, sl] = sh[t, sl] + r0[t, sl] + r1[t, sl]
                    return 0
                lax.fori_loop(0, _D // 128, body, 0)
            pltpu.sync_copy(sh, out_hbm.at[pl.ds(tbase + c * half, half)])

    return k(shp, ysp, slot_flat)


@jax.jit
def kernel(hidden_states, router_weight, gate_up_proj, down_proj,
           gate_w, up_w, down_w, shared_gate_w):
    s, b, d = hidden_states.shape
    x = hidden_states.reshape(-1, d)

    probs, slot, st, sw, bexp = _dispatch(x, router_weight)
    xs = _sc_gather(x, st.reshape(_L))
    ys = _grouped_mm(bexp.reshape(64), xs, gate_up_proj, down_proj,
                     sw.reshape(_L, 1))
    shared = _shared_expert(x, gate_w, up_w, down_w, shared_gate_w)
    out = _sc_combine(shared, ys, slot.reshape(_P))

    return out.astype(jnp.float32).reshape(s, b, d), probs
